# Initial kernel scaffold; baseline (speedup 1.0000x reference)
import jax
import jax.numpy as jnp
from jax.experimental import pallas as pl


def kernel(tfidf_arr, embedding):
    raise NotImplementedError("write your pallas kernel here")


# speedup vs baseline: 10.5582x; 10.5582x over previous
import functools

import jax
import jax.numpy as jnp
from jax import lax
from jax.experimental import pallas as pl
from jax.experimental.pallas import tpu as pltpu
from jax.experimental.pallas import tpu_sc as plsc

TOP_K = 200
L = 16
NBINS = 8192
CAP = 512
CANDBUF = CAP + L
SELBUF = 256
H_UNROLL = 8
P2_UNROLL = 4


def _topk_pool_kernel(B, V, D, tfidf_arr, embedding):
    NW = 32
    RPW = B // NW
    NV = V // L
    DV = D // L
    mesh = plsc.VectorSubcoreMesh(core_axis_name="c", subcore_axis_name="s")

    @functools.partial(
        pl.kernel,
        mesh=mesh,
        out_type=jax.ShapeDtypeStruct((B, D), jnp.float32),
        compiler_params=pltpu.CompilerParams(needs_layout_passes=False),
        scratch_types=[
            pltpu.VMEM((V,), jnp.float32),
            pltpu.VMEM((NBINS,), jnp.float32),
            pltpu.VMEM((CANDBUF,), jnp.float32),
            pltpu.VMEM((CANDBUF,), jnp.int32),
            pltpu.VMEM((SELBUF,), jnp.float32),
            pltpu.VMEM((SELBUF,), jnp.int32),
            pltpu.VMEM((2, 128), jnp.int32),
            pltpu.VMEM((128, D), jnp.float32),
            pltpu.SemaphoreType.DMA,
        ],
    )
    def body(tf_hbm, emb_hbm, out_hbm, row_v, hist_v, cval_v, cidx_v,
             selw_v, self_v, selg_v, rows_v, sem):
        wid = lax.axis_index("s") * 2 + lax.axis_index("c")
        kf = jnp.float32(TOP_K)
        ones = jnp.ones((L,), jnp.float32)
        neg1 = jnp.full((L,), -1.0, jnp.float32)
        zeros_f = jnp.zeros((L,), jnp.float32)
        zeros_i = jnp.zeros((L,), jnp.int32)
        lane_iota = lax.iota(jnp.int32, L)
        scale = jnp.float32(NBINS)

        def bin_of(v):
            b = (v * scale).astype(jnp.int32)
            return jnp.minimum(jnp.maximum(b, 0), NBINS - 1)

        def do_row(r_local, carry):
            r = wid * RPW + r_local
            pltpu.sync_copy(tf_hbm.at[r], row_v)

            def zero_hist(i, c):
                hist_v[pl.ds(i * L, L)] = zeros_f
                return c
            lax.fori_loop(0, NBINS // L, zero_hist, 0)

            def hist_step(i, c):
                for u in range(H_UNROLL):
                    v = row_v[pl.ds((i * H_UNROLL + u) * L, L)]
                    plsc.addupdate_scatter(hist_v, [bin_of(v)], ones)
                return c
            lax.fori_loop(0, NV // H_UNROLL, hist_step, 0)
            for q in range((NV // H_UNROLL) * H_UNROLL, NV):
                v = row_v[pl.ds(q * L, L)]
                plsc.addupdate_scatter(hist_v, [bin_of(v)], ones)

            def wcond(st):
                _, above = st
                return above < kf

            def wbody(st):
                j, above = st
                s = jnp.sum(hist_v[pl.ds(j * L, L)])
                return (j - 1, above + s)

            jend, above_end = lax.while_loop(
                wcond, wbody, (jnp.int32(NBINS // L - 1), jnp.float32(0.0)))
            jc = jend + 1
            h = hist_v[pl.ds(jc * L, L)]
            above_prev = above_end - jnp.sum(h)
            sfx = lax.rev(plsc.cumsum(lax.rev(h, (0,))), (0,))
            in_top = (above_prev + sfx) >= kf
            b_star = jc * L + jnp.sum(in_top.astype(jnp.int32)) - 1

            def fill_cand(i, c):
                cval_v[pl.ds(i * L, L)] = neg1
                return c
            lax.fori_loop(0, CANDBUF // L, fill_cand, 0)

            def collect_one(q, ptr):
                v = row_v[pl.ds(q * L, L)]
                msk = bin_of(v) >= b_star
                ptr_c = jnp.minimum(ptr, CAP)
                plsc.store_compressed(cval_v.at[pl.ds(ptr_c, L)], v, mask=msk)
                plsc.store_compressed(
                    cidx_v.at[pl.ds(ptr_c, L)], q * L + lane_iota, mask=msk)
                return ptr_c + jnp.sum(msk.astype(jnp.int32))

            def p2_step(i, ptr):
                for u in range(P2_UNROLL):
                    ptr = collect_one(i * P2_UNROLL + u, ptr)
                return ptr
            ptr_main = lax.fori_loop(0, NV // P2_UNROLL, p2_step, jnp.int32(0))
            for q in range((NV // P2_UNROLL) * P2_UNROLL, NV):
                ptr_main = collect_one(jnp.int32(q), ptr_main)

            def count_ge(t):
                def cg(q, acc):
                    bits = plsc.bitcast(cval_v[pl.ds(q * L, L)], jnp.int32)
                    return acc + jnp.sum((bits >= t).astype(jnp.int32))
                return lax.fori_loop(0, CANDBUF // L, cg, jnp.int32(0))

            def bs_step(_, st):
                lo, hi = st
                mid = lo + ((hi - lo + 1) >> 1)
                take = count_ge(mid) >= TOP_K
                return (jnp.where(take, mid, lo), jnp.where(take, hi, mid - 1))

            u_bits, _ = lax.fori_loop(
                0, 31, bs_step, (jnp.int32(0), jnp.int32(0x7F7FFFFF)))
            n_gt = count_ge(u_bits + 1)
            n_tie_skip = (count_ge(u_bits) - n_gt) - (TOP_K - n_gt)

            def fill_sel(i, c):
                selw_v[pl.ds(i * L, L)] = zeros_f
                self_v[pl.ds(i * L, L)] = zeros_i
                return c
            lax.fori_loop(0, SELBUF // L, fill_sel, 0)

            def sel_step(q, st):
                ptr2, ties = st
                v = cval_v[pl.ds(q * L, L)]
                ids = cidx_v[pl.ds(q * L, L)]
                bits = plsc.bitcast(v, jnp.int32)
                gt = bits > u_bits
                tie = bits == u_bits
                trank = ties + plsc.cumsum(tie.astype(jnp.int32))
                inc = gt | (tie & (trank > n_tie_skip))
                plsc.store_compressed(selw_v.at[pl.ds(ptr2, L)], v, mask=inc)
                plsc.store_compressed(self_v.at[pl.ds(ptr2, L)], ids, mask=inc)
                return (ptr2 + jnp.sum(inc.astype(jnp.int32)),
                        ties + jnp.sum(tie.astype(jnp.int32)))
            lax.fori_loop(0, CANDBUF // L, sel_step,
                          (jnp.int32(0), jnp.int32(0)))

            for chunk in range(2):
                for col in range(128 // L):
                    selg_v[chunk, pl.ds(col * L, L)] = (
                        self_v[pl.ds(chunk * 128 + col * L, L)])

            accs = tuple(jnp.zeros((L,), jnp.float32) for _ in range(DV))
            for chunk in range(2):
                pltpu.async_copy(emb_hbm.at[selg_v.at[chunk]], rows_v, sem).wait()

                def acc_step(k2, a):
                    w = plsc.load_gather(
                        selw_v, [jnp.full((L,), chunk * 128 + k2, jnp.int32)])
                    return tuple(a[d] + w * rows_v[k2, pl.ds(d * L, L)]
                                 for d in range(DV))
                accs = lax.fori_loop(0, 128, acc_step, accs)

            inv_k = jnp.float32(1.0 / TOP_K)
            for d in range(DV):
                selw_v[pl.ds(d * L, L)] = accs[d] * inv_k
            pltpu.sync_copy(selw_v.at[pl.ds(0, D)], out_hbm.at[r])
            return carry

        lax.fori_loop(0, RPW, do_row, 0)

    return body(tfidf_arr, embedding)


@jax.jit
def kernel(tfidf_arr, embedding):
    B, V = tfidf_arr.shape
    _, D = embedding.shape
    return _topk_pool_kernel(B, V, D, tfidf_arr, embedding)
